# baseline (speedup 1.0000x reference)
import functools

import jax
import jax.numpy as jnp
from jax import lax
from jax.experimental import pallas as pl
from jax.experimental.pallas import tpu as pltpu
from jax.experimental.pallas import tpu_sc as plsc

BM = 512
CW = 64
BM2 = 1024
G = 192

_BIG = 3.0e38
_MASKVAL = 1e10
_MAXI = 2**30


def _knn_body(bss_ref, bcs_ref, psx, psy, psz, bs_row,
              px, py, pz, b_col,
              i0o, i1o, i2o, w0o, w1o, w2o):
    i = pl.program_id(0)
    n = px.shape[0] - CW
    bs_lo = bss_ref[i * BM]
    bs_hi = bss_ref[i * BM + BM - 1]

    ax = psx[...]
    ay = psy[...]
    az = psz[...]
    bsv = bs_row[...]

    def bisect(pred):
        def step(_, ab):
            a, b = ab
            mid = (a + b) // 2
            go_right = pred(bcs_ref[mid])
            return (jnp.where(go_right, mid + 1, a),
                    jnp.where(go_right, b, mid))
        return lax.fori_loop(0, 12, step, (jnp.int32(0), jnp.int32(n)))[0]

    lo = bisect(lambda v: v < bs_lo)
    hi = bisect(lambda v: v <= bs_hi)
    lo8 = (lo // 8) * 8
    nwin = (hi - lo8 + CW - 1) // CW

    init = (jnp.full((1, BM), _BIG, jnp.float32),
            jnp.full((1, BM), _BIG, jnp.float32),
            jnp.full((1, BM), _BIG, jnp.float32),
            jnp.full((1, BM), -1, jnp.int32), jnp.full((1, BM), -2, jnp.int32),
            jnp.full((1, BM), -3, jnp.int32))

    def chunk_step(c, carry):
        v1, v2, v3, x1, x2, x3 = carry
        off = lo8 + c * CW
        sl = pl.ds(off, CW)
        dx = px[sl, 0:1] - ax
        dy = py[sl, 0:1] - ay
        dz = pz[sl, 0:1] - az
        d = dx * dx + dy * dy
        d = d + dz * dz
        d = jnp.where(b_col[sl, 0:1] != bsv, _MASKVAL, d)
        row = lax.broadcasted_iota(jnp.int32, (CW, BM), 0)
        cv, ci = [], []
        for k in range(3):
            m = jnp.min(d, axis=0, keepdims=True)
            li = jnp.min(jnp.where(d == m, row, _MAXI), axis=0,
                         keepdims=True)
            cv.append(m)
            ci.append(li + off)
            if k < 2:
                d = jnp.where(row == li, _BIG, d)
        vs = [v1, v2, v3] + cv
        xs = [x1, x2, x3] + ci
        out_v, out_i = [], []
        for _ in range(3):
            m = functools.reduce(jnp.minimum, vs)
            mi = functools.reduce(
                jnp.minimum,
                [jnp.where(v == m, xx, _MAXI) for v, xx in zip(vs, xs)])
            out_v.append(m)
            out_i.append(mi)
            vs = [jnp.where(xx == mi, _BIG, v) for v, xx in zip(vs, xs)]
        return tuple(out_v) + tuple(out_i)

    v1, v2, v3, x1, x2, x3 = lax.fori_loop(0, nwin, chunk_step, init)
    degen = v1 >= jnp.float32(_MASKVAL)
    v1 = jnp.where(degen, _MASKVAL, v1)
    v2 = jnp.where(degen, _MASKVAL, v2)
    v3 = jnp.where(degen, _MASKVAL, v3)
    x1 = jnp.where(degen, 0, x1)
    x2 = jnp.where(degen, 1, x2)
    x3 = jnp.where(degen, 2, x3)
    x1 = jnp.minimum(x1, n - 1)
    x2 = jnp.minimum(x2, n - 1)
    x3 = jnp.minimum(x3, n - 1)
    i0o[...] = x1
    i1o[...] = x2
    i2o[...] = x3
    w0o[...] = 1.0 / jnp.clip(v1, 1e-16, None)
    w1o[...] = 1.0 / jnp.clip(v2, 1e-16, None)
    w2o[...] = 1.0 / jnp.clip(v3, 1e-16, None)


def _knn_topk(pos, batch, pos_skip, batch_skip):
    n = pos.shape[0]
    m = pos_skip.shape[0]
    grid = (m // BM,)
    padf = jnp.zeros((CW,), jnp.float32)
    padb = jnp.full((CW,), -1, jnp.int32)
    colp = lambda a, j: jnp.concatenate([a[:, j], padf]).reshape(-1, 1)
    row = lambda a, j: a[:, j].reshape(1, -1)
    smem = pl.BlockSpec(memory_space=pltpu.SMEM)
    rowspec = pl.BlockSpec((1, BM), lambda i: (0, i))
    colspec = pl.BlockSpec((n + CW, 1), lambda i: (0, 0))
    outspec = pl.BlockSpec((1, BM), lambda i: (0, i))
    out_sd = [jax.ShapeDtypeStruct((1, m), jnp.int32)] * 3 + \
             [jax.ShapeDtypeStruct((1, m), jnp.float32)] * 3
    return pl.pallas_call(
        _knn_body,
        grid=grid,
        in_specs=[smem, smem, rowspec, rowspec, rowspec, rowspec,
                  colspec, colspec, colspec, colspec],
        out_specs=[outspec] * 6,
        out_shape=out_sd,
    )(batch_skip, batch,
      row(pos_skip, 0), row(pos_skip, 1), row(pos_skip, 2),
      batch_skip.reshape(1, -1),
      colp(pos, 0), colp(pos, 1), colp(pos, 2),
      jnp.concatenate([batch, padb]).reshape(-1, 1))


def _gather_sc(x, idx_flat):
    total = idx_flat.shape[0]
    d = x.shape[1]
    nw = 32
    rows_per_tile = total // nw
    n_groups = rows_per_tile // G
    mesh = plsc.VectorSubcoreMesh(core_axis_name="c", subcore_axis_name="s")

    @functools.partial(
        pl.kernel,
        out_type=jax.ShapeDtypeStruct((total, d), jnp.float32),
        mesh=mesh,
        scratch_types=[
            pltpu.VMEM((rows_per_tile,), jnp.int32),
            pltpu.VMEM((G, d), jnp.float32),
            pltpu.VMEM((G, d), jnp.float32),
            pltpu.SemaphoreType.DMA,
            pltpu.SemaphoreType.DMA,
            pltpu.SemaphoreType.DMA,
            pltpu.SemaphoreType.DMA,
        ],
    )
    def gather_kernel(x_hbm, idx_hbm, out_hbm, idx_v, r0, r1,
                      gs0, gs1, os0, os1):
        wid = lax.axis_index("s") * 2 + lax.axis_index("c")
        base = wid * rows_per_tile
        pltpu.sync_copy(idx_hbm.at[pl.ds(base, rows_per_tile)], idx_v)
        rows = [r0, r1]
        gsem = [gs0, gs1]
        osem = [os0, os1]
        gcopy = [None, None]
        ocopy = [None, None]
        for g in range(n_groups):
            b = g % 2
            if g >= 2:
                ocopy[b].wait()
            gcopy[b] = pltpu.async_copy(
                x_hbm.at[idx_v.at[pl.ds(g * G, G)]], rows[b], gsem[b])
            if g >= 1:
                bp = (g - 1) % 2
                gcopy[bp].wait()
                ocopy[bp] = pltpu.async_copy(
                    rows[bp], out_hbm.at[pl.ds(base + (g - 1) * G, G)],
                    osem[bp])
        last = (n_groups - 1) % 2
        gcopy[last].wait()
        pltpu.async_copy(
            rows[last], out_hbm.at[pl.ds(base + (n_groups - 1) * G, G)],
            osem[last]).wait()
        ocopy[(n_groups - 2) % 2].wait()

    return gather_kernel(x, idx_flat)


def _mlp_body(g0, g1, g2, w0r, w1r, w2r, xs, W1a, W1b, b1r, W2r, b2r, out):
    w0 = w0r[...]
    w1 = w1r[...]
    w2 = w2r[...]
    num = w0 * g0[0] + w1 * g1[0]
    num = num + w2 * g2[0]
    den = w0 + w1
    den = den + w2
    h = num / den
    dot = functools.partial(jnp.dot, precision=lax.Precision.DEFAULT,
                            preferred_element_type=jnp.float32)
    a = dot(h, W1a[...]) + dot(xs[...], W1b[...]) + b1r[...]
    a = jnp.maximum(a, 0.0)
    out[...] = dot(a, W2r[...]) + b2r[...]


def _mlp(g3, w0, w1, w2, x_skip, W1, b1, W2, b2):
    m = x_skip.shape[0]
    d_in = g3.shape[2]
    d_skip = x_skip.shape[1]
    d_hid = W1.shape[1]
    d_out = W2.shape[1]
    grid = (m // BM2,)
    gspec = lambda j: pl.BlockSpec((1, BM2, d_in), lambda i, j=j: (j, i, 0))
    colspec = pl.BlockSpec((BM2, 1), lambda i: (i, 0))
    full = lambda r, c: pl.BlockSpec((r, c), lambda i: (0, 0))
    return pl.pallas_call(
        _mlp_body,
        grid=grid,
        in_specs=[gspec(0), gspec(1), gspec(2), colspec, colspec, colspec,
                  pl.BlockSpec((BM2, d_skip), lambda i: (i, 0)),
                  full(d_in, d_hid), full(d_skip, d_hid), full(1, d_hid),
                  full(d_hid, d_out), full(1, d_out)],
        out_specs=pl.BlockSpec((BM2, d_out), lambda i: (i, 0)),
        out_shape=jax.ShapeDtypeStruct((m, d_out), jnp.float32),
    )(g3, g3, g3, w0, w1, w2, x_skip,
      W1[:d_in], W1[d_in:], b1.reshape(1, -1), W2, b2.reshape(1, -1))


def kernel(x, pos, batch, x_skip, pos_skip, batch_skip, W1, b1, W2, b2):
    batch = batch.astype(jnp.int32)
    batch_skip = batch_skip.astype(jnp.int32)
    m = pos_skip.shape[0]
    i0, i1, i2, w0, w1, w2 = _knn_topk(pos, batch, pos_skip, batch_skip)
    idx_flat = jnp.concatenate([i0, i1, i2], axis=0).reshape(-1)
    g = _gather_sc(x, idx_flat)
    g3 = g.reshape(3, m, x.shape[1])
    return _mlp(g3, w0.reshape(m, 1), w1.reshape(m, 1), w2.reshape(m, 1),
                x_skip, W1, b1, W2, b2)


# speedup vs baseline: 1.1118x; 1.1118x over previous
import functools

import jax
import jax.numpy as jnp
from jax import lax
from jax.experimental import pallas as pl
from jax.experimental.pallas import tpu as pltpu
from jax.experimental.pallas import tpu_sc as plsc

BM = 512
CW = 128
BM2 = 1024
G = 192

_BIG = 3.0e38
_MASKVAL = 1e10
_MAXI = 2**30


def _knn_body(bss_ref, bcs_ref, psx, psy, psz, bs_row,
              px, py, pz, b_col,
              i0o, i1o, i2o, w0o, w1o, w2o):
    i = pl.program_id(0)
    n = px.shape[0] - CW
    bs_lo = bss_ref[i * BM]
    bs_hi = bss_ref[i * BM + BM - 1]

    ax = psx[...]
    ay = psy[...]
    az = psz[...]
    bsv = bs_row[...]

    def bisect(pred):
        def step(_, ab):
            a, b = ab
            mid = (a + b) // 2
            go_right = pred(bcs_ref[mid])
            return (jnp.where(go_right, mid + 1, a),
                    jnp.where(go_right, b, mid))
        return lax.fori_loop(0, 12, step, (jnp.int32(0), jnp.int32(n)))[0]

    lo = bisect(lambda v: v < bs_lo)
    hi = bisect(lambda v: v <= bs_hi)
    lo8 = (lo // 8) * 8
    nwin = (hi - lo8 + CW - 1) // CW

    init = (jnp.full((1, BM), _BIG, jnp.float32),
            jnp.full((1, BM), _BIG, jnp.float32),
            jnp.full((1, BM), _BIG, jnp.float32),
            jnp.full((1, BM), -1, jnp.int32), jnp.full((1, BM), -2, jnp.int32),
            jnp.full((1, BM), -3, jnp.int32))

    def chunk_step(c, carry):
        v1, v2, v3, x1, x2, x3 = carry
        off = lo8 + c * CW
        sl = pl.ds(off, CW)
        dx = px[sl, 0:1] - ax
        dy = py[sl, 0:1] - ay
        dz = pz[sl, 0:1] - az
        d = dx * dx + dy * dy
        d = d + dz * dz
        d = jnp.where(b_col[sl, 0:1] != bsv, _MASKVAL, d)
        row = lax.broadcasted_iota(jnp.int32, (CW, BM), 0)
        cv, ci = [], []
        for k in range(3):
            m = jnp.min(d, axis=0, keepdims=True)
            li = jnp.min(jnp.where(d == m, row, _MAXI), axis=0,
                         keepdims=True)
            cv.append(m)
            ci.append(li + off)
            if k < 2:
                d = jnp.where(row == li, _BIG, d)
        vs = [v1, v2, v3] + cv
        xs = [x1, x2, x3] + ci
        out_v, out_i = [], []
        for _ in range(3):
            m = functools.reduce(jnp.minimum, vs)
            mi = functools.reduce(
                jnp.minimum,
                [jnp.where(v == m, xx, _MAXI) for v, xx in zip(vs, xs)])
            out_v.append(m)
            out_i.append(mi)
            vs = [jnp.where(xx == mi, _BIG, v) for v, xx in zip(vs, xs)]
        return tuple(out_v) + tuple(out_i)

    v1, v2, v3, x1, x2, x3 = lax.fori_loop(0, nwin, chunk_step, init)
    degen = v1 >= jnp.float32(_MASKVAL)
    v1 = jnp.where(degen, _MASKVAL, v1)
    v2 = jnp.where(degen, _MASKVAL, v2)
    v3 = jnp.where(degen, _MASKVAL, v3)
    x1 = jnp.where(degen, 0, x1)
    x2 = jnp.where(degen, 1, x2)
    x3 = jnp.where(degen, 2, x3)
    x1 = jnp.minimum(x1, n - 1)
    x2 = jnp.minimum(x2, n - 1)
    x3 = jnp.minimum(x3, n - 1)
    i0o[...] = x1
    i1o[...] = x2
    i2o[...] = x3
    w0o[...] = 1.0 / jnp.clip(v1, 1e-16, None)
    w1o[...] = 1.0 / jnp.clip(v2, 1e-16, None)
    w2o[...] = 1.0 / jnp.clip(v3, 1e-16, None)


def _knn_topk(pos, batch, pos_skip, batch_skip):
    n = pos.shape[0]
    m = pos_skip.shape[0]
    grid = (m // BM,)
    padf = jnp.zeros((CW,), jnp.float32)
    padb = jnp.full((CW,), -1, jnp.int32)
    colp = lambda a, j: jnp.concatenate([a[:, j], padf]).reshape(-1, 1)
    row = lambda a, j: a[:, j].reshape(1, -1)
    smem = pl.BlockSpec(memory_space=pltpu.SMEM)
    rowspec = pl.BlockSpec((1, BM), lambda i: (0, i))
    colspec = pl.BlockSpec((n + CW, 1), lambda i: (0, 0))
    outspec = pl.BlockSpec((1, BM), lambda i: (0, i))
    out_sd = [jax.ShapeDtypeStruct((1, m), jnp.int32)] * 3 + \
             [jax.ShapeDtypeStruct((1, m), jnp.float32)] * 3
    return pl.pallas_call(
        _knn_body,
        grid=grid,
        in_specs=[smem, smem, rowspec, rowspec, rowspec, rowspec,
                  colspec, colspec, colspec, colspec],
        out_specs=[outspec] * 6,
        out_shape=out_sd,
    )(batch_skip, batch,
      row(pos_skip, 0), row(pos_skip, 1), row(pos_skip, 2),
      batch_skip.reshape(1, -1),
      colp(pos, 0), colp(pos, 1), colp(pos, 2),
      jnp.concatenate([batch, padb]).reshape(-1, 1))


def _gather_sc(x, idx_flat):
    total = idx_flat.shape[0]
    d = x.shape[1]
    nw = 32
    rows_per_tile = total // nw
    n_groups = rows_per_tile // G
    mesh = plsc.VectorSubcoreMesh(core_axis_name="c", subcore_axis_name="s")

    @functools.partial(
        pl.kernel,
        out_type=jax.ShapeDtypeStruct((total, d), jnp.float32),
        mesh=mesh,
        scratch_types=[
            pltpu.VMEM((rows_per_tile,), jnp.int32),
            pltpu.VMEM((G, d), jnp.float32),
            pltpu.VMEM((G, d), jnp.float32),
            pltpu.SemaphoreType.DMA,
            pltpu.SemaphoreType.DMA,
            pltpu.SemaphoreType.DMA,
            pltpu.SemaphoreType.DMA,
        ],
    )
    def gather_kernel(x_hbm, idx_hbm, out_hbm, idx_v, r0, r1,
                      gs0, gs1, os0, os1):
        wid = lax.axis_index("s") * 2 + lax.axis_index("c")
        base = wid * rows_per_tile
        pltpu.sync_copy(idx_hbm.at[pl.ds(base, rows_per_tile)], idx_v)
        rows = [r0, r1]
        gsem = [gs0, gs1]
        osem = [os0, os1]
        gcopy = [None, None]
        ocopy = [None, None]
        for g in range(n_groups):
            b = g % 2
            if g >= 2:
                ocopy[b].wait()
            gcopy[b] = pltpu.async_copy(
                x_hbm.at[idx_v.at[pl.ds(g * G, G)]], rows[b], gsem[b])
            if g >= 1:
                bp = (g - 1) % 2
                gcopy[bp].wait()
                ocopy[bp] = pltpu.async_copy(
                    rows[bp], out_hbm.at[pl.ds(base + (g - 1) * G, G)],
                    osem[bp])
        last = (n_groups - 1) % 2
        gcopy[last].wait()
        pltpu.async_copy(
            rows[last], out_hbm.at[pl.ds(base + (n_groups - 1) * G, G)],
            osem[last]).wait()
        ocopy[(n_groups - 2) % 2].wait()

    return gather_kernel(x, idx_flat)


def _mlp_body(g0, g1, g2, w0r, w1r, w2r, xs, W1a, W1b, b1r, W2r, b2r, out):
    w0 = w0r[...]
    w1 = w1r[...]
    w2 = w2r[...]
    num = w0 * g0[0] + w1 * g1[0]
    num = num + w2 * g2[0]
    den = w0 + w1
    den = den + w2
    h = num / den
    dot = functools.partial(jnp.dot, precision=lax.Precision.DEFAULT,
                            preferred_element_type=jnp.float32)
    a = dot(h, W1a[...]) + dot(xs[...], W1b[...]) + b1r[...]
    a = jnp.maximum(a, 0.0)
    out[...] = dot(a, W2r[...]) + b2r[...]


def _mlp(g3, w0, w1, w2, x_skip, W1, b1, W2, b2):
    m = x_skip.shape[0]
    d_in = g3.shape[2]
    d_skip = x_skip.shape[1]
    d_hid = W1.shape[1]
    d_out = W2.shape[1]
    grid = (m // BM2,)
    gspec = lambda j: pl.BlockSpec((1, BM2, d_in), lambda i, j=j: (j, i, 0))
    colspec = pl.BlockSpec((BM2, 1), lambda i: (i, 0))
    full = lambda r, c: pl.BlockSpec((r, c), lambda i: (0, 0))
    return pl.pallas_call(
        _mlp_body,
        grid=grid,
        in_specs=[gspec(0), gspec(1), gspec(2), colspec, colspec, colspec,
                  pl.BlockSpec((BM2, d_skip), lambda i: (i, 0)),
                  full(d_in, d_hid), full(d_skip, d_hid), full(1, d_hid),
                  full(d_hid, d_out), full(1, d_out)],
        out_specs=pl.BlockSpec((BM2, d_out), lambda i: (i, 0)),
        out_shape=jax.ShapeDtypeStruct((m, d_out), jnp.float32),
    )(g3, g3, g3, w0, w1, w2, x_skip,
      W1[:d_in], W1[d_in:], b1.reshape(1, -1), W2, b2.reshape(1, -1))


def kernel(x, pos, batch, x_skip, pos_skip, batch_skip, W1, b1, W2, b2):
    batch = batch.astype(jnp.int32)
    batch_skip = batch_skip.astype(jnp.int32)
    m = pos_skip.shape[0]
    i0, i1, i2, w0, w1, w2 = _knn_topk(pos, batch, pos_skip, batch_skip)
    idx_flat = jnp.concatenate([i0, i1, i2], axis=0).reshape(-1)
    g = _gather_sc(x, idx_flat)
    g3 = g.reshape(3, m, x.shape[1])
    return _mlp(g3, w0.reshape(m, 1), w1.reshape(m, 1), w2.reshape(m, 1),
                x_skip, W1, b1, W2, b2)
